# Initial kernel scaffold; baseline (speedup 1.0000x reference)
import jax
import jax.numpy as jnp
from jax.experimental import pallas as pl


def kernel(x, edge_index, edge_attr, node_enc_w, node_enc_b, edge_enc_w, edge_enc_b, dec_w, dec_b, msg_w, msg_b, upd_w, upd_b):
    raise NotImplementedError("write your pallas kernel here")


# speedup vs baseline: 3.1931x; 3.1931x over previous
import functools

import jax
import jax.numpy as jnp
from jax import lax
from jax.experimental import pallas as pl
from jax.experimental.pallas import tpu as pltpu
from jax.experimental.pallas import tpu_sc as plsc

N = 10000
E = 320000
D = 128
DE = 16
NC = 2
NS = 16
NW = NC * NS
EPW = E // NW
CH = 80
NSTEP = EPW // CH
RBASE = 624
ZR = 128
ZCOPIES = 5
LANES = 16

_mesh = plsc.VectorSubcoreMesh(core_axis_name="c", subcore_axis_name="s")


@functools.partial(
    pl.kernel,
    out_type=jax.ShapeDtypeStruct((NC, N, D), jnp.float32),
    mesh=_mesh,
    scratch_types=(
        pltpu.VMEM_SHARED((N, D), jnp.float32),
        pltpu.VMEM((CH,), jnp.int32),
        pltpu.VMEM((CH,), jnp.int32),
        pltpu.VMEM((CH, D), jnp.float32),
        pltpu.VMEM((CH, D), jnp.float32),
        pltpu.VMEM((ZR, D), jnp.float32),
        pltpu.SemaphoreType.DMA,
    ),
)
def _sc_layer(h_hbm, c_hbm, src_hbm, dst_hbm, out_hbm,
              aggr, sidx, didx, gbuf, cbuf, zbuf, gsem):
    c = lax.axis_index("c")
    s = lax.axis_index("s")
    wid = s * NC + c

    zero = jnp.zeros((LANES,), jnp.float32)

    def zrow(e, carry):
        for j in range(D // LANES):
            zbuf[e, pl.ds(j * LANES, LANES)] = zero
        return carry

    lax.fori_loop(0, ZR, zrow, None)
    for k in range(ZCOPIES):
        pltpu.sync_copy(zbuf, aggr.at[pl.ds(s * RBASE + k * ZR, ZR)])
    plsc.subcore_barrier()

    def step(i, carry):
        base = wid * EPW + i * CH
        pltpu.sync_copy(src_hbm.at[pl.ds(base, CH)], sidx)
        pltpu.sync_copy(dst_hbm.at[pl.ds(base, CH)], didx)
        cp = pltpu.async_copy(h_hbm.at[sidx], gbuf, gsem)
        pltpu.sync_copy(c_hbm.at[pl.ds(base, CH)], cbuf)
        cp.wait()

        def edge(e, inner):
            for j in range(D // LANES):
                sl = pl.ds(j * LANES, LANES)
                gbuf[e, sl] = jnp.maximum(gbuf[e, sl] + cbuf[e, sl], 0.0)
            return inner

        lax.fori_loop(0, CH, edge, None)
        pltpu.sync_copy(gbuf, aggr.at[didx], add=True)
        return carry

    lax.fori_loop(0, NSTEP, step, None)

    plsc.subcore_barrier()
    for k in range(ZCOPIES):
        r0 = s * RBASE + k * ZR
        pltpu.sync_copy(aggr.at[pl.ds(r0, ZR)], out_hbm.at[c, pl.ds(r0, ZR)])


@functools.partial(
    pl.kernel,
    out_type=jax.ShapeDtypeStruct((E, DE), jnp.float32),
    mesh=_mesh,
    scratch_types=(
        pltpu.VMEM((CH,), jnp.int32),
        pltpu.VMEM((CH,), jnp.int32),
        pltpu.VMEM((CH, DE), jnp.float32),
        pltpu.VMEM((CH, DE), jnp.float32),
        pltpu.SemaphoreType.DMA,
        pltpu.SemaphoreType.DMA,
    ),
    compiler_params=pltpu.CompilerParams(use_tc_tiling_on_sc=False),
)
def _sc_decode(p_hbm, q_hbm, src_hbm, dst_hbm, out_hbm,
               sidx, didx, pbuf, qbuf, psem, qsem):
    c = lax.axis_index("c")
    s = lax.axis_index("s")
    wid = s * NC + c

    def step(i, carry):
        base = wid * EPW + i * CH
        pltpu.sync_copy(src_hbm.at[pl.ds(base, CH)], sidx)
        pltpu.sync_copy(dst_hbm.at[pl.ds(base, CH)], didx)
        cp1 = pltpu.async_copy(p_hbm.at[sidx], pbuf, psem)
        cp2 = pltpu.async_copy(q_hbm.at[didx], qbuf, qsem)
        cp1.wait()
        cp2.wait()

        def edge(e, inner):
            sl = pl.ds(0, LANES)
            pbuf[e, sl] = pbuf[e, sl] + qbuf[e, sl]
            return inner

        lax.fori_loop(0, CH, edge, None)
        pltpu.sync_copy(pbuf, out_hbm.at[pl.ds(base, CH)])
        return carry

    lax.fori_loop(0, NSTEP, step, None)


def _enc_body(x_ref, w_ref, b_ref, w1_ref, hb_ref, nf_ref, h_ref):
    nf = jnp.dot(x_ref[...], w_ref[...], preferred_element_type=jnp.float32)
    nf = nf + b_ref[...]
    nf_ref[...] = nf
    h_ref[...] = jnp.dot(nf, w1_ref[...],
                         preferred_element_type=jnp.float32) + hb_ref[...]


def _tc_encode(x, w, b, w1, hb):
    return pl.pallas_call(
        _enc_body,
        out_shape=(jax.ShapeDtypeStruct((N, D), jnp.float32),
                   jax.ShapeDtypeStruct((N, D), jnp.float32)),
    )(x, w, b, w1, hb)


CBLK = 4000


def _cmul_body(ea_ref, w_ref, c0_ref, c1_ref, c2_ref):
    r = jnp.dot(ea_ref[...], w_ref[...], preferred_element_type=jnp.float32)
    c0_ref[...] = r[:, 0:D]
    c1_ref[...] = r[:, D:2 * D]
    c2_ref[...] = r[:, 2 * D:3 * D]


def _tc_cmul(edge_attr, w2cat):
    grid = (E // CBLK,)
    return pl.pallas_call(
        _cmul_body,
        grid=grid,
        in_specs=[
            pl.BlockSpec((CBLK, DE), lambda i: (i, 0)),
            pl.BlockSpec((DE, 3 * D), lambda i: (0, 0)),
        ],
        out_specs=(pl.BlockSpec((CBLK, D), lambda i: (i, 0)),) * 3,
        out_shape=(jax.ShapeDtypeStruct((E, D), jnp.float32),) * 3,
    )(edge_attr, w2cat)


def _upd_body(nf_ref, a0_ref, a1_ref, uwa_ref, uwb_ref, ub_ref, w1_ref, hb_ref,
              nf2_ref, h_ref):
    nf = nf_ref[...]
    ag = a0_ref[...] + a1_ref[...]
    u = jnp.dot(nf, uwa_ref[...], preferred_element_type=jnp.float32)
    u = u + jnp.dot(ag, uwb_ref[...], preferred_element_type=jnp.float32)
    u = jnp.maximum(u + ub_ref[...], 0.0)
    nf2 = nf + u
    nf2_ref[...] = nf2
    h_ref[...] = jnp.dot(nf2, w1_ref[...],
                         preferred_element_type=jnp.float32) + hb_ref[...]


def _tc_update(nf, a0, a1, uwa, uwb, ub, w1, hb):
    return pl.pallas_call(
        _upd_body,
        out_shape=(jax.ShapeDtypeStruct((N, D), jnp.float32),
                   jax.ShapeDtypeStruct((N, D), jnp.float32)),
    )(nf, a0, a1, uwa, uwb, ub, w1, hb)


def _updlast_body(nf_ref, a0_ref, a1_ref, uwa_ref, uwb_ref, ub_ref,
                  dwa_ref, dwb_ref, db_ref, p_ref, q_ref):
    nf = nf_ref[...]
    ag = a0_ref[...] + a1_ref[...]
    u = jnp.dot(nf, uwa_ref[...], preferred_element_type=jnp.float32)
    u = u + jnp.dot(ag, uwb_ref[...], preferred_element_type=jnp.float32)
    u = jnp.maximum(u + ub_ref[...], 0.0)
    nf2 = nf + u
    p_ref[...] = jnp.dot(nf2, dwa_ref[...],
                         preferred_element_type=jnp.float32) + db_ref[...]
    q_ref[...] = jnp.dot(nf2, dwb_ref[...], preferred_element_type=jnp.float32)


def _tc_update_last(nf, a0, a1, uwa, uwb, ub, dwa, dwb, db):
    return pl.pallas_call(
        _updlast_body,
        out_shape=(jax.ShapeDtypeStruct((N, DE), jnp.float32),
                   jax.ShapeDtypeStruct((N, DE), jnp.float32)),
    )(nf, a0, a1, uwa, uwb, ub, dwa, dwb, db)


def kernel(x, edge_index, edge_attr, node_enc_w, node_enc_b, edge_enc_w,
           edge_enc_b, dec_w, dec_b, msg_w, msg_b, upd_w, upd_b):
    src = edge_index[0].astype(jnp.int32)
    dst = edge_index[1].astype(jnp.int32)

    w1 = msg_w[:, :D, :]
    w2 = jnp.einsum("ef,lfm->lem", edge_enc_w,
                    msg_w[:, D:, :])
    hb = jnp.einsum("f,lfm->lm", edge_enc_b,
                    msg_w[:, D:, :]) + msg_b
    w2cat = jnp.concatenate([w2[0], w2[1], w2[2]], axis=1)
    uwa = upd_w[:, :D, :]
    uwb = upd_w[:, D:, :]

    nf, h = _tc_encode(x, node_enc_w, node_enc_b[None], w1[0], hb[0][None])
    c0, c1, c2 = _tc_cmul(edge_attr, w2cat)
    cs = (c0, c1, c2)

    for l in range(2):
        a = _sc_layer(h, cs[l], src, dst)
        nf, h = _tc_update(nf, a[0], a[1], uwa[l], uwb[l], upd_b[l][None],
                           w1[l + 1], hb[l + 1][None])
    a = _sc_layer(h, cs[2], src, dst)
    p, q = _tc_update_last(nf, a[0], a[1], uwa[2], uwb[2], upd_b[2][None],
                           dec_w[:D], dec_w[D:], dec_b[None])
    return _sc_decode(p, q, src, dst)


# speedup vs baseline: 5.6714x; 1.7761x over previous
import functools

import jax
import jax.numpy as jnp
from jax import lax
from jax.experimental import pallas as pl
from jax.experimental.pallas import tpu as pltpu
from jax.experimental.pallas import tpu_sc as plsc

N = 10000
E = 320000
D = 128
DE = 16
NC = 2
NS = 16
NW = NC * NS
EPW = E // NW
CHL = 40
NSTEPL = EPW // CHL
CHD = 80
NSTEPD = EPW // CHD
RBASE = 624
ZR = 128
ZCOPIES = 5
LANES = 16

_mesh = plsc.VectorSubcoreMesh(core_axis_name="c", subcore_axis_name="s")


NBUF = 3


def _run_pipeline(nstep, process):
    process(0, 0, True, True, True)
    full = nstep - 4
    triples = full // 3

    def triple(k, carry):
        i = 3 * k + 1
        process(i, 1, True, True, False)
        process(i + 1, 2, True, True, False)
        process(i + 2, 0, True, True, False)
        return carry

    lax.fori_loop(0, triples, triple, None)
    for i in range(3 * triples + 1, nstep - 3):
        process(i, i % 3, True, True, False)
    process(nstep - 3, (nstep - 3) % 3, False, True, False)
    process(nstep - 2, (nstep - 2) % 3, False, False, False)
    process(nstep - 1, (nstep - 1) % 3, False, False, False)


@functools.partial(
    pl.kernel,
    out_type=jax.ShapeDtypeStruct((NC, N, D), jnp.float32),
    mesh=_mesh,
    scratch_types=(
        pltpu.VMEM_SHARED((N, D), jnp.float32),
        pltpu.VMEM((NBUF, CHL), jnp.int32),
        pltpu.VMEM((NBUF, CHL), jnp.int32),
        pltpu.VMEM((NBUF, CHL, D), jnp.float32),
        pltpu.VMEM((NBUF, CHL, D // 2), jnp.int32),
    ) + (pltpu.SemaphoreType.DMA,) * (5 * NBUF),
    compiler_params=pltpu.CompilerParams(needs_layout_passes=False),
)
def _sc_layer(h_hbm, c_hbm, src_hbm, dst_hbm, out_hbm,
              aggr, sidx, didx, gbuf, cbuf, *sems):
    isem = sems[0:NBUF]
    dsem = sems[NBUF:2 * NBUF]
    csem = sems[2 * NBUF:3 * NBUF]
    gsem = sems[3 * NBUF:4 * NBUF]
    ssem = sems[4 * NBUF:5 * NBUF]

    c = lax.axis_index("c")
    s = lax.axis_index("s")
    wid = s * NC + c
    ebase = wid * EPW

    zero = jnp.zeros((LANES,), jnp.float32)

    def zrow(e, carry):
        for j in range(D // LANES):
            gbuf[0, e, pl.ds(j * LANES, LANES)] = zero
        return carry

    lax.fori_loop(0, CHL, zrow, None)
    for k in range(16):
        pltpu.sync_copy(gbuf.at[0], aggr.at[pl.ds(s * RBASE + k * CHL, CHL)])
    plsc.subcore_barrier()

    def issue_sidx(i, b):
        pltpu.async_copy(src_hbm.at[pl.ds(ebase + i * CHL, CHL)],
                         sidx.at[b], isem[b])

    def issue_didx(i, b):
        pltpu.async_copy(dst_hbm.at[pl.ds(ebase + i * CHL, CHL)],
                         didx.at[b], dsem[b])

    def issue_c(i, b):
        pltpu.async_copy(c_hbm.at[pl.ds(ebase + i * CHL, CHL)],
                         cbuf.at[b], csem[b])

    def issue_gather(b):
        pltpu.async_copy(h_hbm.at[sidx.at[b]], gbuf.at[b], gsem[b])

    def wait_gather(b):
        pltpu.make_async_copy(h_hbm.at[sidx.at[b]], gbuf.at[b],
                              gsem[b]).wait()

    def wait_lin(i, b, hbm, buf, sem):
        pltpu.make_async_copy(hbm.at[pl.ds(ebase + i * CHL, CHL)],
                              buf.at[b], sem[b]).wait()

    def issue_scatter(b):
        pltpu.async_copy(gbuf.at[b], aggr.at[didx.at[b]], ssem[b], add=True)

    def wait_scatter(b):
        pltpu.make_async_copy(gbuf.at[b], aggr.at[didx.at[b]],
                              ssem[b]).wait()

    def process(i, b, deep_pf, pf, first):
        wait_gather(b)
        wait_lin(i, b, c_hbm, cbuf, csem)
        if deep_pf:
            issue_sidx(i + 3, b)

        def edge(e, carry):
            for j in range(D // 32):
                pair = plsc.bitcast(cbuf[b, e, pl.ds(LANES * j, LANES)],
                                    jnp.bfloat16)
                lo, hi = plsc.unpack(pair,
                                     format=plsc.PackFormat.INTERLEAVED)
                sl0 = pl.ds(32 * j, LANES)
                sl1 = pl.ds(32 * j + LANES, LANES)
                gbuf[b, e, sl0] = jnp.maximum(gbuf[b, e, sl0] + lo, 0.0)
                gbuf[b, e, sl1] = jnp.maximum(gbuf[b, e, sl1] + hi, 0.0)
            return carry

        lax.fori_loop(0, CHL, edge, None)
        wait_lin(i, b, dst_hbm, didx, dsem)
        issue_scatter(b)
        if pf:
            b2 = (b + 2) % NBUF
            if not first:
                wait_scatter(b2)
            issue_didx(i + 2, b2)
            issue_c(i + 2, b2)
            pltpu.make_async_copy(src_hbm.at[pl.ds(ebase, CHL)],
                                  sidx.at[b2], isem[b2]).wait()
            issue_gather(b2)

    for i in (0, 1):
        issue_sidx(i, i)
        issue_didx(i, i)
        issue_c(i, i)
        pltpu.make_async_copy(src_hbm.at[pl.ds(ebase, CHL)],
                              sidx.at[i], isem[i]).wait()
        issue_gather(i)
    issue_sidx(2, 2)

    _run_pipeline(NSTEPL, process)

    for b in range(NBUF):
        wait_scatter(b)

    plsc.subcore_barrier()
    for k in range(ZCOPIES):
        r0 = s * RBASE + k * ZR
        pltpu.sync_copy(aggr.at[pl.ds(r0, ZR)], out_hbm.at[c, pl.ds(r0, ZR)])


@functools.partial(
    pl.kernel,
    out_type=jax.ShapeDtypeStruct((E, DE), jnp.float32),
    mesh=_mesh,
    scratch_types=(
        pltpu.VMEM((NBUF, CHD), jnp.int32),
        pltpu.VMEM((NBUF, CHD), jnp.int32),
        pltpu.VMEM((NBUF, CHD, DE), jnp.float32),
        pltpu.VMEM((NBUF, CHD, DE), jnp.float32),
    ) + (pltpu.SemaphoreType.DMA,) * (5 * NBUF),
    compiler_params=pltpu.CompilerParams(use_tc_tiling_on_sc=False),
)
def _sc_decode(p_hbm, q_hbm, src_hbm, dst_hbm, out_hbm,
               sidx, didx, pbuf, qbuf, *sems):
    isem = sems[0:NBUF]
    dsem = sems[NBUF:2 * NBUF]
    psem = sems[2 * NBUF:3 * NBUF]
    qsem = sems[3 * NBUF:4 * NBUF]
    wsem = sems[4 * NBUF:5 * NBUF]

    c = lax.axis_index("c")
    s = lax.axis_index("s")
    ebase = (s * NC + c) * EPW

    def issue_idx(i, b):
        pltpu.async_copy(src_hbm.at[pl.ds(ebase + i * CHD, CHD)],
                         sidx.at[b], isem[b])
        pltpu.async_copy(dst_hbm.at[pl.ds(ebase + i * CHD, CHD)],
                         didx.at[b], dsem[b])

    def wait_idx(b):
        pltpu.make_async_copy(src_hbm.at[pl.ds(ebase, CHD)],
                              sidx.at[b], isem[b]).wait()
        pltpu.make_async_copy(dst_hbm.at[pl.ds(ebase, CHD)],
                              didx.at[b], dsem[b]).wait()

    def issue_gathers(b):
        pltpu.async_copy(p_hbm.at[sidx.at[b]], pbuf.at[b], psem[b])
        pltpu.async_copy(q_hbm.at[didx.at[b]], qbuf.at[b], qsem[b])

    def wait_gathers(b):
        pltpu.make_async_copy(p_hbm.at[sidx.at[b]], pbuf.at[b],
                              psem[b]).wait()
        pltpu.make_async_copy(q_hbm.at[didx.at[b]], qbuf.at[b],
                              qsem[b]).wait()

    def wait_write(b):
        pltpu.make_async_copy(pbuf.at[b], out_hbm.at[pl.ds(ebase, CHD)],
                              wsem[b]).wait()

    def process(i, b, deep_pf, pf, first):
        wait_gathers(b)
        if deep_pf:
            issue_idx(i + 3, b)

        def edge(e, carry):
            sl = pl.ds(0, LANES)
            pbuf[b, e, sl] = pbuf[b, e, sl] + qbuf[b, e, sl]
            return carry

        lax.fori_loop(0, CHD, edge, None)
        pltpu.async_copy(pbuf.at[b], out_hbm.at[pl.ds(ebase + i * CHD, CHD)],
                         wsem[b])
        if pf:
            b2 = (b + 2) % NBUF
            if not first:
                wait_write(b2)
            wait_idx(b2)
            issue_gathers(b2)

    for i in (0, 1):
        issue_idx(i, i)
        wait_idx(i)
        issue_gathers(i)
    issue_idx(2, 2)

    _run_pipeline(NSTEPD, process)

    for b in range(NBUF):
        wait_write(b)


def _enc_body(x_ref, w_ref, b_ref, w1_ref, hb_ref, nf_ref, h_ref):
    nf = jnp.dot(x_ref[...], w_ref[...], preferred_element_type=jnp.float32)
    nf = nf + b_ref[...]
    nf_ref[...] = nf
    h_ref[...] = jnp.dot(nf, w1_ref[...],
                         preferred_element_type=jnp.float32) + hb_ref[...]


def _tc_encode(x, w, b, w1, hb):
    return pl.pallas_call(
        _enc_body,
        out_shape=(jax.ShapeDtypeStruct((N, D), jnp.float32),
                   jax.ShapeDtypeStruct((N, D), jnp.float32)),
    )(x, w, b, w1, hb)


CBLK = 4000


def _cmul_body(ea_ref, we_ref, wo_ref, c0_ref, c1_ref, c2_ref):
    ea = ea_ref[...]
    re = jnp.dot(ea, we_ref[...], preferred_element_type=jnp.float32)
    ro = jnp.dot(ea, wo_ref[...], preferred_element_type=jnp.float32)
    ue = jax.lax.bitcast_convert_type(re.astype(jnp.bfloat16),
                                      jnp.uint16).astype(jnp.uint32)
    uo = jax.lax.bitcast_convert_type(ro.astype(jnp.bfloat16),
                                      jnp.uint16).astype(jnp.uint32)
    packed = jax.lax.bitcast_convert_type(ue | (uo << 16), jnp.int32)
    h = D // 2
    c0_ref[...] = packed[:, 0:h]
    c1_ref[...] = packed[:, h:2 * h]
    c2_ref[...] = packed[:, 2 * h:3 * h]


def _tc_cmul(edge_attr, w_even, w_odd):
    grid = (E // CBLK,)
    h = D // 2
    return pl.pallas_call(
        _cmul_body,
        grid=grid,
        in_specs=[
            pl.BlockSpec((CBLK, DE), lambda i: (i, 0)),
            pl.BlockSpec((DE, 3 * h), lambda i: (0, 0)),
            pl.BlockSpec((DE, 3 * h), lambda i: (0, 0)),
        ],
        out_specs=(pl.BlockSpec((CBLK, h), lambda i: (i, 0)),) * 3,
        out_shape=(jax.ShapeDtypeStruct((E, h), jnp.int32),) * 3,
    )(edge_attr, w_even, w_odd)


def _upd_body(nf_ref, a0_ref, a1_ref, uwa_ref, uwb_ref, ub_ref, w1_ref, hb_ref,
              nf2_ref, h_ref):
    nf = nf_ref[...]
    ag = a0_ref[...] + a1_ref[...]
    u = jnp.dot(nf, uwa_ref[...], preferred_element_type=jnp.float32)
    u = u + jnp.dot(ag, uwb_ref[...], preferred_element_type=jnp.float32)
    u = jnp.maximum(u + ub_ref[...], 0.0)
    nf2 = nf + u
    nf2_ref[...] = nf2
    h_ref[...] = jnp.dot(nf2, w1_ref[...],
                         preferred_element_type=jnp.float32) + hb_ref[...]


def _tc_update(nf, a0, a1, uwa, uwb, ub, w1, hb):
    return pl.pallas_call(
        _upd_body,
        out_shape=(jax.ShapeDtypeStruct((N, D), jnp.float32),
                   jax.ShapeDtypeStruct((N, D), jnp.float32)),
    )(nf, a0, a1, uwa, uwb, ub, w1, hb)


def _updlast_body(nf_ref, a0_ref, a1_ref, uwa_ref, uwb_ref, ub_ref,
                  dwa_ref, dwb_ref, db_ref, p_ref, q_ref):
    nf = nf_ref[...]
    ag = a0_ref[...] + a1_ref[...]
    u = jnp.dot(nf, uwa_ref[...], preferred_element_type=jnp.float32)
    u = u + jnp.dot(ag, uwb_ref[...], preferred_element_type=jnp.float32)
    u = jnp.maximum(u + ub_ref[...], 0.0)
    nf2 = nf + u
    p_ref[...] = jnp.dot(nf2, dwa_ref[...],
                         preferred_element_type=jnp.float32) + db_ref[...]
    q_ref[...] = jnp.dot(nf2, dwb_ref[...], preferred_element_type=jnp.float32)


def _tc_update_last(nf, a0, a1, uwa, uwb, ub, dwa, dwb, db):
    return pl.pallas_call(
        _updlast_body,
        out_shape=(jax.ShapeDtypeStruct((N, DE), jnp.float32),
                   jax.ShapeDtypeStruct((N, DE), jnp.float32)),
    )(nf, a0, a1, uwa, uwb, ub, dwa, dwb, db)


def kernel(x, edge_index, edge_attr, node_enc_w, node_enc_b, edge_enc_w,
           edge_enc_b, dec_w, dec_b, msg_w, msg_b, upd_w, upd_b):
    src = edge_index[0].astype(jnp.int32)
    dst = edge_index[1].astype(jnp.int32)

    w1 = msg_w[:, :D, :]
    w2 = jnp.einsum("ef,lfm->lem", edge_enc_w,
                    msg_w[:, D:, :])
    hb = jnp.einsum("f,lfm->lm", edge_enc_b,
                    msg_w[:, D:, :]) + msg_b
    w2cat = jnp.concatenate([w2[0], w2[1], w2[2]], axis=1)
    col = jnp.arange(3 * D // 2)
    t32, k16 = col // LANES * 32, col % LANES
    w_even = w2cat[:, t32 + k16]
    w_odd = w2cat[:, t32 + LANES + k16]
    uwa = upd_w[:, :D, :]
    uwb = upd_w[:, D:, :]

    nf, h = _tc_encode(x, node_enc_w, node_enc_b[None], w1[0], hb[0][None])
    c0, c1, c2 = _tc_cmul(edge_attr, w_even, w_odd)
    cs = (c0, c1, c2)

    for l in range(2):
        a = _sc_layer(h, cs[l], src, dst)
        nf, h = _tc_update(nf, a[0], a[1], uwa[l], uwb[l], upd_b[l][None],
                           w1[l + 1], hb[l + 1][None])
    a = _sc_layer(h, cs[2], src, dst)
    p, q = _tc_update_last(nf, a[0], a[1], uwa[2], uwb[2], upd_b[2][None],
                           dec_w[:D], dec_w[D:], dec_b[None])
    return _sc_decode(p, q, src, dst)


# speedup vs baseline: 6.6196x; 1.1672x over previous
import functools

import jax
import jax.numpy as jnp
from jax import lax
from jax.experimental import pallas as pl
from jax.experimental.pallas import tpu as pltpu
from jax.experimental.pallas import tpu_sc as plsc

N = 10000
E = 320000
D = 128
DE = 16
NC = 2
NS = 16
NW = NC * NS
EPW = E // NW
CHL = 40
NSTEPL = EPW // CHL
CHD = 80
NSTEPD = EPW // CHD
RBASE = 624
ZR = 128
ZCOPIES = 5
LANES = 16

_mesh = plsc.VectorSubcoreMesh(core_axis_name="c", subcore_axis_name="s")


NBUF = 3


def _run_pipeline(nstep, process):
    process(0, 0, True, True, True)
    full = nstep - 4
    triples = full // 3

    def triple(k, carry):
        i = 3 * k + 1
        process(i, 1, True, True, False)
        process(i + 1, 2, True, True, False)
        process(i + 2, 0, True, True, False)
        return carry

    lax.fori_loop(0, triples, triple, None)
    for i in range(3 * triples + 1, nstep - 3):
        process(i, i % 3, True, True, False)
    process(nstep - 3, (nstep - 3) % 3, False, True, False)
    process(nstep - 2, (nstep - 2) % 3, False, False, False)
    process(nstep - 1, (nstep - 1) % 3, False, False, False)


@functools.partial(
    pl.kernel,
    out_type=(jax.ShapeDtypeStruct((N, D), jnp.float32),
              jax.ShapeDtypeStruct((N, D), jnp.float32)),
    mesh=_mesh,
    scratch_types=(
        pltpu.VMEM_SHARED((N, D), jnp.float32),
        pltpu.VMEM((NBUF, CHL), jnp.int32),
        pltpu.VMEM((NBUF, CHL), jnp.int32),
        pltpu.VMEM((NBUF, CHL, D), jnp.float32),
        pltpu.VMEM((NBUF, CHL, D // 2), jnp.int32),
    ) + (pltpu.SemaphoreType.DMA,) * (5 * NBUF),
    compiler_params=pltpu.CompilerParams(needs_layout_passes=False),
)
def _sc_layer(h_hbm, c_hbm, src_hbm, dst_hbm, out0_hbm, out1_hbm,
              aggr, sidx, didx, gbuf, cbuf, *sems):
    isem = sems[0:NBUF]
    dsem = sems[NBUF:2 * NBUF]
    csem = sems[2 * NBUF:3 * NBUF]
    gsem = sems[3 * NBUF:4 * NBUF]
    ssem = sems[4 * NBUF:5 * NBUF]

    c = lax.axis_index("c")
    s = lax.axis_index("s")
    wid = s * NC + c
    ebase = wid * EPW

    zero = jnp.zeros((LANES,), jnp.float32)

    def zrow(e, carry):
        for j in range(D // LANES):
            gbuf[0, e, pl.ds(j * LANES, LANES)] = zero
        return carry

    lax.fori_loop(0, CHL, zrow, None)
    for k in range(16):
        pltpu.sync_copy(gbuf.at[0], aggr.at[pl.ds(s * RBASE + k * CHL, CHL)])
    plsc.subcore_barrier()

    def issue_sidx(i, b):
        pltpu.async_copy(src_hbm.at[pl.ds(ebase + i * CHL, CHL)],
                         sidx.at[b], isem[b])

    def issue_didx(i, b):
        pltpu.async_copy(dst_hbm.at[pl.ds(ebase + i * CHL, CHL)],
                         didx.at[b], dsem[b])

    def issue_c(i, b):
        pltpu.async_copy(c_hbm.at[pl.ds(ebase + i * CHL, CHL)],
                         cbuf.at[b], csem[b])

    def issue_gather(b):
        pltpu.async_copy(h_hbm.at[sidx.at[b]], gbuf.at[b], gsem[b])

    def wait_gather(b):
        pltpu.make_async_copy(h_hbm.at[sidx.at[b]], gbuf.at[b],
                              gsem[b]).wait()

    def wait_lin(i, b, hbm, buf, sem):
        pltpu.make_async_copy(hbm.at[pl.ds(ebase + i * CHL, CHL)],
                              buf.at[b], sem[b]).wait()

    def issue_scatter(b):
        pltpu.async_copy(gbuf.at[b], aggr.at[didx.at[b]], ssem[b], add=True)

    def wait_scatter(b):
        pltpu.make_async_copy(gbuf.at[b], aggr.at[didx.at[b]],
                              ssem[b]).wait()

    def process(i, b, deep_pf, pf, first):
        wait_gather(b)
        wait_lin(i, b, c_hbm, cbuf, csem)
        if deep_pf:
            issue_sidx(i + 3, b)

        def edge(e, carry):
            for j in range(D // 32):
                pair = plsc.bitcast(cbuf[b, e, pl.ds(LANES * j, LANES)],
                                    jnp.bfloat16)
                lo, hi = plsc.unpack(pair,
                                     format=plsc.PackFormat.INTERLEAVED)
                sl0 = pl.ds(32 * j, LANES)
                sl1 = pl.ds(32 * j + LANES, LANES)
                gbuf[b, e, sl0] = jnp.maximum(gbuf[b, e, sl0] + lo, 0.0)
                gbuf[b, e, sl1] = jnp.maximum(gbuf[b, e, sl1] + hi, 0.0)
            return carry

        lax.fori_loop(0, CHL, edge, None)
        wait_lin(i, b, dst_hbm, didx, dsem)
        issue_scatter(b)
        if pf:
            b2 = (b + 2) % NBUF
            if not first:
                wait_scatter(b2)
            issue_didx(i + 2, b2)
            issue_c(i + 2, b2)
            pltpu.make_async_copy(src_hbm.at[pl.ds(ebase, CHL)],
                                  sidx.at[b2], isem[b2]).wait()
            issue_gather(b2)

    for i in (0, 1):
        issue_sidx(i, i)
        issue_didx(i, i)
        issue_c(i, i)
        pltpu.make_async_copy(src_hbm.at[pl.ds(ebase, CHL)],
                              sidx.at[i], isem[i]).wait()
        issue_gather(i)
    issue_sidx(2, 2)

    _run_pipeline(NSTEPL, process)

    for b in range(NBUF):
        wait_scatter(b)

    plsc.subcore_barrier()
    for k in range(ZCOPIES):
        r0 = s * RBASE + k * ZR

        @pl.when(c == 0)
        def _():
            pltpu.sync_copy(aggr.at[pl.ds(r0, ZR)], out0_hbm.at[pl.ds(r0, ZR)])

        @pl.when(c == 1)
        def _():
            pltpu.sync_copy(aggr.at[pl.ds(r0, ZR)], out1_hbm.at[pl.ds(r0, ZR)])


CHD8 = CHD // 8


@functools.partial(
    pl.kernel,
    out_type=jax.ShapeDtypeStruct((E // 8, D), jnp.float32),
    mesh=_mesh,
    scratch_types=(
        pltpu.VMEM((NBUF, CHD), jnp.int32),
        pltpu.VMEM((NBUF, CHD), jnp.int32),
        pltpu.VMEM((NBUF, CHD, DE), jnp.float32),
        pltpu.VMEM((NBUF, CHD, DE), jnp.float32),
        pltpu.VMEM((NBUF, CHD8, D), jnp.float32),
    ) + (pltpu.SemaphoreType.DMA,) * (5 * NBUF),
    compiler_params=pltpu.CompilerParams(use_tc_tiling_on_sc=False),
)
def _sc_decode(p_hbm, q_hbm, src_hbm, dst_hbm, out_hbm,
               sidx, didx, pbuf, qbuf, wbuf, *sems):
    isem = sems[0:NBUF]
    dsem = sems[NBUF:2 * NBUF]
    psem = sems[2 * NBUF:3 * NBUF]
    qsem = sems[3 * NBUF:4 * NBUF]
    wsem = sems[4 * NBUF:5 * NBUF]

    c = lax.axis_index("c")
    s = lax.axis_index("s")
    ebase = (s * NC + c) * EPW
    obase = (s * NC + c) * (EPW // 8)

    def issue_idx(i, b):
        pltpu.async_copy(src_hbm.at[pl.ds(ebase + i * CHD, CHD)],
                         sidx.at[b], isem[b])
        pltpu.async_copy(dst_hbm.at[pl.ds(ebase + i * CHD, CHD)],
                         didx.at[b], dsem[b])

    def wait_idx(b):
        pltpu.make_async_copy(src_hbm.at[pl.ds(ebase, CHD)],
                              sidx.at[b], isem[b]).wait()
        pltpu.make_async_copy(dst_hbm.at[pl.ds(ebase, CHD)],
                              didx.at[b], dsem[b]).wait()

    def issue_gathers(b):
        pltpu.async_copy(p_hbm.at[sidx.at[b]], pbuf.at[b], psem[b])
        pltpu.async_copy(q_hbm.at[didx.at[b]], qbuf.at[b], qsem[b])

    def wait_gathers(b):
        pltpu.make_async_copy(p_hbm.at[sidx.at[b]], pbuf.at[b],
                              psem[b]).wait()
        pltpu.make_async_copy(q_hbm.at[didx.at[b]], qbuf.at[b],
                              qsem[b]).wait()

    def wait_write(b):
        pltpu.make_async_copy(wbuf.at[b], out_hbm.at[pl.ds(obase, CHD8)],
                              wsem[b]).wait()

    def process(i, b, deep_pf, pf, first):
        wait_gathers(b)
        if deep_pf:
            issue_idx(i + 3, b)

        def grp(g, carry):
            for sub in range(8):
                e = 8 * g + sub
                wbuf[b, g, pl.ds(LANES * sub, LANES)] = (
                    pbuf[b, e, pl.ds(0, LANES)] + qbuf[b, e, pl.ds(0, LANES)])
            return carry

        lax.fori_loop(0, CHD8, grp, None)
        pltpu.async_copy(wbuf.at[b],
                         out_hbm.at[pl.ds(obase + i * CHD8, CHD8)], wsem[b])
        if pf:
            b2 = (b + 2) % NBUF
            if not first:
                wait_write(b2)
            wait_idx(b2)
            issue_gathers(b2)

    for i in (0, 1):
        issue_idx(i, i)
        wait_idx(i)
        issue_gathers(i)
    issue_idx(2, 2)

    _run_pipeline(NSTEPD, process)

    for b in range(NBUF):
        wait_write(b)


def _enc_body(x_ref, w_ref, b_ref, w1_ref, hb_ref, nf_ref, h_ref):
    nf = jnp.dot(x_ref[...], w_ref[...], preferred_element_type=jnp.float32)
    nf = nf + b_ref[...]
    nf_ref[...] = nf
    h_ref[...] = jnp.dot(nf, w1_ref[...],
                         preferred_element_type=jnp.float32) + hb_ref[...]


def _tc_encode(x, w, b, w1, hb):
    return pl.pallas_call(
        _enc_body,
        out_shape=(jax.ShapeDtypeStruct((N, D), jnp.float32),
                   jax.ShapeDtypeStruct((N, D), jnp.float32)),
    )(x, w, b, w1, hb)


CBLK = 6400


def _rne_bf16_bits(x):
    u = jax.lax.bitcast_convert_type(x, jnp.uint32)
    return u + jnp.uint32(0x7FFF) + ((u >> 16) & jnp.uint32(1))


def _cmul_body(eat_ref, we_ref, wo_ref, c_ref):
    dn = (((0,), (0,)), ((), ()))
    re = jax.lax.dot_general(eat_ref[...], we_ref[...], dn,
                             preferred_element_type=jnp.float32)
    ro = jax.lax.dot_general(eat_ref[...], wo_ref[...], dn,
                             preferred_element_type=jnp.float32)
    packed = (_rne_bf16_bits(re) >> 16) | (_rne_bf16_bits(ro)
                                           & jnp.uint32(0xFFFF0000))
    c_ref[...] = jax.lax.bitcast_convert_type(packed, jnp.int32)


def _tc_cmul_layer(ea_t, w_even, w_odd):
    h = D // 2
    return pl.pallas_call(
        _cmul_body,
        grid=(E // CBLK,),
        in_specs=[
            pl.BlockSpec((DE, CBLK), lambda i: (0, i)),
            pl.BlockSpec((DE, h), lambda i: (0, 0)),
            pl.BlockSpec((DE, h), lambda i: (0, 0)),
        ],
        out_specs=pl.BlockSpec((CBLK, h), lambda i: (i, 0)),
        out_shape=jax.ShapeDtypeStruct((E, h), jnp.int32),
    )(ea_t, w_even, w_odd)


def _upd_body(nf_ref, a0_ref, a1_ref, uwa_ref, uwb_ref, ub_ref, w1_ref, hb_ref,
              nf2_ref, h_ref):
    nf = nf_ref[...]
    ag = a0_ref[...] + a1_ref[...]
    u = jnp.dot(nf, uwa_ref[...], preferred_element_type=jnp.float32)
    u = u + jnp.dot(ag, uwb_ref[...], preferred_element_type=jnp.float32)
    u = jnp.maximum(u + ub_ref[...], 0.0)
    nf2 = nf + u
    nf2_ref[...] = nf2
    h_ref[...] = jnp.dot(nf2, w1_ref[...],
                         preferred_element_type=jnp.float32) + hb_ref[...]


def _tc_update(nf, a0, a1, uwa, uwb, ub, w1, hb):
    return pl.pallas_call(
        _upd_body,
        out_shape=(jax.ShapeDtypeStruct((N, D), jnp.float32),
                   jax.ShapeDtypeStruct((N, D), jnp.float32)),
    )(nf, a0, a1, uwa, uwb, ub, w1, hb)


def _updlast_body(nf_ref, a0_ref, a1_ref, uwa_ref, uwb_ref, ub_ref,
                  dwa_ref, dwb_ref, db_ref, p_ref, q_ref):
    nf = nf_ref[...]
    ag = a0_ref[...] + a1_ref[...]
    u = jnp.dot(nf, uwa_ref[...], preferred_element_type=jnp.float32)
    u = u + jnp.dot(ag, uwb_ref[...], preferred_element_type=jnp.float32)
    u = jnp.maximum(u + ub_ref[...], 0.0)
    nf2 = nf + u
    p_ref[...] = jnp.dot(nf2, dwa_ref[...],
                         preferred_element_type=jnp.float32) + db_ref[...]
    q_ref[...] = jnp.dot(nf2, dwb_ref[...], preferred_element_type=jnp.float32)


def _tc_update_last(nf, a0, a1, uwa, uwb, ub, dwa, dwb, db):
    return pl.pallas_call(
        _updlast_body,
        out_shape=(jax.ShapeDtypeStruct((N, DE), jnp.float32),
                   jax.ShapeDtypeStruct((N, DE), jnp.float32)),
    )(nf, a0, a1, uwa, uwb, ub, dwa, dwb, db)


def kernel(x, edge_index, edge_attr, node_enc_w, node_enc_b, edge_enc_w,
           edge_enc_b, dec_w, dec_b, msg_w, msg_b, upd_w, upd_b):
    src = edge_index[0].astype(jnp.int32)
    dst = edge_index[1].astype(jnp.int32)

    w1 = msg_w[:, :D, :]
    w2 = jnp.einsum("ef,lfm->lem", edge_enc_w,
                    msg_w[:, D:, :])
    hb = jnp.einsum("f,lfm->lm", edge_enc_b,
                    msg_w[:, D:, :]) + msg_b
    w2cat = jnp.concatenate([w2[0], w2[1], w2[2]], axis=1)
    col = jnp.arange(3 * D // 2)
    t32, k16 = col // LANES * 32, col % LANES
    w_even = w2cat[:, t32 + k16]
    w_odd = w2cat[:, t32 + LANES + k16]
    uwa = upd_w[:, :D, :]
    uwb = upd_w[:, D:, :]

    nf, h = _tc_encode(x, node_enc_w, node_enc_b[None], w1[0], hb[0][None])
    ea_t = edge_attr.T
    hw = 3 * D // 2
    cs = [_tc_cmul_layer(ea_t, w_even[:, l * hw // 3:(l + 1) * hw // 3],
                         w_odd[:, l * hw // 3:(l + 1) * hw // 3])
          for l in range(3)]

    for l in range(2):
        a0, a1 = _sc_layer(h, cs[l], src, dst)
        nf, h = _tc_update(nf, a0, a1, uwa[l], uwb[l], upd_b[l][None],
                           w1[l + 1], hb[l + 1][None])
    a0, a1 = _sc_layer(h, cs[2], src, dst)
    p, q = _tc_update_last(nf, a0, a1, uwa[2], uwb[2], upd_b[2][None],
                           dec_w[:D], dec_w[D:], dec_b[None])
    return _sc_decode(p, q, src, dst).reshape(E, DE)


# speedup vs baseline: 7.4475x; 1.1251x over previous
import functools

import jax
import jax.numpy as jnp
from jax import lax
from jax.experimental import pallas as pl
from jax.experimental.pallas import tpu as pltpu
from jax.experimental.pallas import tpu_sc as plsc

N = 10000
E = 320000
D = 128
DE = 16
NC = 2
NS = 16
NW = NC * NS
EPW = E // NW
CHL = 40
NSTEPL = EPW // CHL
CHD = 80
NSTEPD = EPW // CHD
RBASE = 624
ZR = 128
ZCOPIES = 5
LANES = 16

_mesh = plsc.VectorSubcoreMesh(core_axis_name="c", subcore_axis_name="s")


NBUF = 3


def _run_pipeline(nstep, process):
    process(0, 0, True, True, True)
    full = nstep - 4
    triples = full // 3

    def triple(k, carry):
        i = 3 * k + 1
        process(i, 1, True, True, False)
        process(i + 1, 2, True, True, False)
        process(i + 2, 0, True, True, False)
        return carry

    lax.fori_loop(0, triples, triple, None)
    for i in range(3 * triples + 1, nstep - 3):
        process(i, i % 3, True, True, False)
    process(nstep - 3, (nstep - 3) % 3, False, True, False)
    process(nstep - 2, (nstep - 2) % 3, False, False, False)
    process(nstep - 1, (nstep - 1) % 3, False, False, False)


@functools.partial(
    pl.kernel,
    out_type=(jax.ShapeDtypeStruct((N, D), jnp.float32),
              jax.ShapeDtypeStruct((N, D), jnp.float32)),
    mesh=_mesh,
    scratch_types=(
        pltpu.VMEM_SHARED((N, D), jnp.float32),
        pltpu.VMEM((NBUF, CHL), jnp.int32),
        pltpu.VMEM((NBUF, CHL), jnp.int32),
        pltpu.VMEM((NBUF, CHL, D), jnp.float32),
        pltpu.VMEM((NBUF, CHL, D // 2), jnp.int32),
    ) + (pltpu.SemaphoreType.DMA,) * (5 * NBUF),
    compiler_params=pltpu.CompilerParams(needs_layout_passes=False),
)
def _sc_layer(h_hbm, c_hbm, src_hbm, dst_hbm, out0_hbm, out1_hbm,
              aggr, sidx, didx, gbuf, cbuf, *sems):
    isem = sems[0:NBUF]
    dsem = sems[NBUF:2 * NBUF]
    csem = sems[2 * NBUF:3 * NBUF]
    gsem = sems[3 * NBUF:4 * NBUF]
    ssem = sems[4 * NBUF:5 * NBUF]

    c = lax.axis_index("c")
    s = lax.axis_index("s")
    wid = s * NC + c
    ebase = wid * EPW

    zero = jnp.zeros((LANES,), jnp.float32)

    def zrow(e, carry):
        for j in range(D // LANES):
            gbuf[0, e, pl.ds(j * LANES, LANES)] = zero
        return carry

    lax.fori_loop(0, CHL, zrow, None)
    for k in range(16):
        pltpu.sync_copy(gbuf.at[0], aggr.at[pl.ds(s * RBASE + k * CHL, CHL)])
    plsc.subcore_barrier()

    def issue_sidx(i, b):
        pltpu.async_copy(src_hbm.at[pl.ds(ebase + i * CHL, CHL)],
                         sidx.at[b], isem[b])

    def issue_didx(i, b):
        pltpu.async_copy(dst_hbm.at[pl.ds(ebase + i * CHL, CHL)],
                         didx.at[b], dsem[b])

    def issue_c(i, b):
        pltpu.async_copy(c_hbm.at[pl.ds(ebase + i * CHL, CHL)],
                         cbuf.at[b], csem[b])

    def issue_gather(b):
        pltpu.async_copy(h_hbm.at[sidx.at[b]], gbuf.at[b], gsem[b])

    def wait_gather(b):
        pltpu.make_async_copy(h_hbm.at[sidx.at[b]], gbuf.at[b],
                              gsem[b]).wait()

    def wait_lin(i, b, hbm, buf, sem):
        pltpu.make_async_copy(hbm.at[pl.ds(ebase + i * CHL, CHL)],
                              buf.at[b], sem[b]).wait()

    def issue_scatter(b):
        pltpu.async_copy(gbuf.at[b], aggr.at[didx.at[b]], ssem[b], add=True)

    def wait_scatter(b):
        pltpu.make_async_copy(gbuf.at[b], aggr.at[didx.at[b]],
                              ssem[b]).wait()

    def process(i, b, deep_pf, pf, first):
        wait_gather(b)
        wait_lin(i, b, c_hbm, cbuf, csem)
        if deep_pf:
            issue_sidx(i + 3, b)

        def edge(e, carry):
            for j in range(D // 32):
                pair = plsc.bitcast(cbuf[b, e, pl.ds(LANES * j, LANES)],
                                    jnp.bfloat16)
                lo, hi = plsc.unpack(pair,
                                     format=plsc.PackFormat.INTERLEAVED)
                sl0 = pl.ds(32 * j, LANES)
                sl1 = pl.ds(32 * j + LANES, LANES)
                gbuf[b, e, sl0] = jnp.maximum(gbuf[b, e, sl0] + lo, 0.0)
                gbuf[b, e, sl1] = jnp.maximum(gbuf[b, e, sl1] + hi, 0.0)
            return carry

        lax.fori_loop(0, CHL, edge, None)
        wait_lin(i, b, dst_hbm, didx, dsem)
        issue_scatter(b)
        if pf:
            b2 = (b + 2) % NBUF
            if not first:
                wait_scatter(b2)
            issue_didx(i + 2, b2)
            issue_c(i + 2, b2)
            pltpu.make_async_copy(src_hbm.at[pl.ds(ebase, CHL)],
                                  sidx.at[b2], isem[b2]).wait()
            issue_gather(b2)

    for i in (0, 1):
        issue_sidx(i, i)
        issue_didx(i, i)
        issue_c(i, i)
        pltpu.make_async_copy(src_hbm.at[pl.ds(ebase, CHL)],
                              sidx.at[i], isem[i]).wait()
        issue_gather(i)
    issue_sidx(2, 2)

    _run_pipeline(NSTEPL, process)

    for b in range(NBUF):
        wait_scatter(b)

    plsc.subcore_barrier()
    for k in range(ZCOPIES):
        r0 = s * RBASE + k * ZR

        @pl.when(c == 0)
        def _():
            pltpu.sync_copy(aggr.at[pl.ds(r0, ZR)], out0_hbm.at[pl.ds(r0, ZR)])

        @pl.when(c == 1)
        def _():
            pltpu.sync_copy(aggr.at[pl.ds(r0, ZR)], out1_hbm.at[pl.ds(r0, ZR)])


CHD8 = CHD // 8


@functools.partial(
    pl.kernel,
    out_type=jax.ShapeDtypeStruct((DE, E), jnp.float32),
    mesh=_mesh,
    scratch_types=(
        pltpu.VMEM((NBUF, CHD), jnp.int32),
        pltpu.VMEM((NBUF, CHD), jnp.int32),
        pltpu.VMEM((NBUF, CHD, DE), jnp.float32),
        pltpu.VMEM((NBUF, CHD, DE), jnp.float32),
        pltpu.VMEM((NBUF, DE, CHD), jnp.float32),
    ) + (pltpu.SemaphoreType.DMA,) * (5 * NBUF),
    compiler_params=pltpu.CompilerParams(use_tc_tiling_on_sc=False,
                                         needs_layout_passes=False),
)
def _sc_decode(p_hbm, q_hbm, src_hbm, dst_hbm, out_hbm,
               sidx, didx, pbuf, qbuf, wbuf, *sems):
    isem = sems[0:NBUF]
    dsem = sems[NBUF:2 * NBUF]
    psem = sems[2 * NBUF:3 * NBUF]
    qsem = sems[3 * NBUF:4 * NBUF]
    wsem = sems[4 * NBUF:5 * NBUF]

    c = lax.axis_index("c")
    s = lax.axis_index("s")
    ebase = (s * NC + c) * EPW
    iota16 = lax.iota(jnp.int32, LANES)

    def issue_idx(i, b):
        pltpu.async_copy(src_hbm.at[pl.ds(ebase + i * CHD, CHD)],
                         sidx.at[b], isem[b])
        pltpu.async_copy(dst_hbm.at[pl.ds(ebase + i * CHD, CHD)],
                         didx.at[b], dsem[b])

    def wait_idx(b):
        pltpu.make_async_copy(src_hbm.at[pl.ds(ebase, CHD)],
                              sidx.at[b], isem[b]).wait()
        pltpu.make_async_copy(dst_hbm.at[pl.ds(ebase, CHD)],
                              didx.at[b], dsem[b]).wait()

    def issue_gathers(b):
        pltpu.async_copy(p_hbm.at[sidx.at[b]], pbuf.at[b], psem[b])
        pltpu.async_copy(q_hbm.at[didx.at[b]], qbuf.at[b], qsem[b])

    def wait_gathers(b):
        pltpu.make_async_copy(p_hbm.at[sidx.at[b]], pbuf.at[b],
                              psem[b]).wait()
        pltpu.make_async_copy(q_hbm.at[didx.at[b]], qbuf.at[b],
                              qsem[b]).wait()

    def wait_write(b):
        pltpu.make_async_copy(wbuf.at[b], out_hbm.at[:, pl.ds(ebase, CHD)],
                              wsem[b]).wait()

    def process(i, b, deep_pf, pf, first):
        wait_gathers(b)
        if deep_pf:
            issue_idx(i + 3, b)

        def edge(e, carry):
            v = pbuf[b, e, pl.ds(0, LANES)] + qbuf[b, e, pl.ds(0, LANES)]
            plsc.store_scatter(wbuf.at[b],
                               [iota16, jnp.full((LANES,), e, jnp.int32)], v)
            return carry

        lax.fori_loop(0, CHD, edge, None)
        pltpu.async_copy(wbuf.at[b],
                         out_hbm.at[:, pl.ds(ebase + i * CHD, CHD)], wsem[b])
        if pf:
            b2 = (b + 2) % NBUF
            if not first:
                wait_write(b2)
            wait_idx(b2)
            issue_gathers(b2)

    for i in (0, 1):
        issue_idx(i, i)
        wait_idx(i)
        issue_gathers(i)
    issue_idx(2, 2)

    _run_pipeline(NSTEPD, process)

    for b in range(NBUF):
        wait_write(b)


def _enc_body(x_ref, w_ref, b_ref, w1_ref, hb_ref, nf_ref, h_ref):
    nf = jnp.dot(x_ref[...], w_ref[...], preferred_element_type=jnp.float32)
    nf = nf + b_ref[...]
    nf_ref[...] = nf
    h_ref[...] = jnp.dot(nf, w1_ref[...],
                         preferred_element_type=jnp.float32) + hb_ref[...]


def _tc_encode(x, w, b, w1, hb):
    return pl.pallas_call(
        _enc_body,
        out_shape=(jax.ShapeDtypeStruct((N, D), jnp.float32),
                   jax.ShapeDtypeStruct((N, D), jnp.float32)),
    )(x, w, b, w1, hb)


CBLK = 6400


def _rhu_bf16_bits(x):
    return jax.lax.bitcast_convert_type(x, jnp.uint32) + jnp.uint32(0x8000)


def _cmul_body(eat_ref, we_ref, wo_ref, c_ref):
    dn = (((0,), (0,)), ((), ()))
    re = jax.lax.dot_general(eat_ref[...], we_ref[...], dn,
                             preferred_element_type=jnp.float32)
    ro = jax.lax.dot_general(eat_ref[...], wo_ref[...], dn,
                             preferred_element_type=jnp.float32)
    packed = (_rhu_bf16_bits(re) >> 16) | (_rhu_bf16_bits(ro)
                                           & jnp.uint32(0xFFFF0000))
    c_ref[...] = jax.lax.bitcast_convert_type(packed, jnp.int32)


def _tc_cmul_layer(ea_t, w_even, w_odd):
    h = D // 2
    return pl.pallas_call(
        _cmul_body,
        grid=(E // CBLK,),
        in_specs=[
            pl.BlockSpec((DE, CBLK), lambda i: (0, i)),
            pl.BlockSpec((DE, h), lambda i: (0, 0)),
            pl.BlockSpec((DE, h), lambda i: (0, 0)),
        ],
        out_specs=pl.BlockSpec((CBLK, h), lambda i: (i, 0)),
        out_shape=jax.ShapeDtypeStruct((E, h), jnp.int32),
    )(ea_t, w_even, w_odd)


def _upd_body(nf_ref, a0_ref, a1_ref, uwa_ref, uwb_ref, ub_ref, w1_ref, hb_ref,
              nf2_ref, h_ref):
    nf = nf_ref[...]
    ag = a0_ref[...] + a1_ref[...]
    u = jnp.dot(nf, uwa_ref[...], preferred_element_type=jnp.float32)
    u = u + jnp.dot(ag, uwb_ref[...], preferred_element_type=jnp.float32)
    u = jnp.maximum(u + ub_ref[...], 0.0)
    nf2 = nf + u
    nf2_ref[...] = nf2
    h_ref[...] = jnp.dot(nf2, w1_ref[...],
                         preferred_element_type=jnp.float32) + hb_ref[...]


def _tc_update(nf, a0, a1, uwa, uwb, ub, w1, hb):
    return pl.pallas_call(
        _upd_body,
        out_shape=(jax.ShapeDtypeStruct((N, D), jnp.float32),
                   jax.ShapeDtypeStruct((N, D), jnp.float32)),
    )(nf, a0, a1, uwa, uwb, ub, w1, hb)


def _updlast_body(nf_ref, a0_ref, a1_ref, uwa_ref, uwb_ref, ub_ref,
                  dwa_ref, dwb_ref, db_ref, p_ref, q_ref):
    nf = nf_ref[...]
    ag = a0_ref[...] + a1_ref[...]
    u = jnp.dot(nf, uwa_ref[...], preferred_element_type=jnp.float32)
    u = u + jnp.dot(ag, uwb_ref[...], preferred_element_type=jnp.float32)
    u = jnp.maximum(u + ub_ref[...], 0.0)
    nf2 = nf + u
    p_ref[...] = jnp.dot(nf2, dwa_ref[...],
                         preferred_element_type=jnp.float32) + db_ref[...]
    q_ref[...] = jnp.dot(nf2, dwb_ref[...], preferred_element_type=jnp.float32)


def _tc_update_last(nf, a0, a1, uwa, uwb, ub, dwa, dwb, db):
    return pl.pallas_call(
        _updlast_body,
        out_shape=(jax.ShapeDtypeStruct((N, DE), jnp.float32),
                   jax.ShapeDtypeStruct((N, DE), jnp.float32)),
    )(nf, a0, a1, uwa, uwb, ub, dwa, dwb, db)


def kernel(x, edge_index, edge_attr, node_enc_w, node_enc_b, edge_enc_w,
           edge_enc_b, dec_w, dec_b, msg_w, msg_b, upd_w, upd_b):
    src = edge_index[0].astype(jnp.int32)
    dst = edge_index[1].astype(jnp.int32)

    w1 = msg_w[:, :D, :]
    w2 = jnp.einsum("ef,lfm->lem", edge_enc_w,
                    msg_w[:, D:, :])
    hb = jnp.einsum("f,lfm->lm", edge_enc_b,
                    msg_w[:, D:, :]) + msg_b
    w2cat = jnp.concatenate([w2[0], w2[1], w2[2]], axis=1)
    col = jnp.arange(3 * D // 2)
    t32, k16 = col // LANES * 32, col % LANES
    w_even = w2cat[:, t32 + k16]
    w_odd = w2cat[:, t32 + LANES + k16]
    uwa = upd_w[:, :D, :]
    uwb = upd_w[:, D:, :]

    nf, h = _tc_encode(x, node_enc_w, node_enc_b[None], w1[0], hb[0][None])
    ea_t = edge_attr.T
    hw = 3 * D // 2
    cs = [_tc_cmul_layer(ea_t, w_even[:, l * hw // 3:(l + 1) * hw // 3],
                         w_odd[:, l * hw // 3:(l + 1) * hw // 3])
          for l in range(3)]

    for l in range(2):
        a0, a1 = _sc_layer(h, cs[l], src, dst)
        nf, h = _tc_update(nf, a0, a1, uwa[l], uwb[l], upd_b[l][None],
                           w1[l + 1], hb[l + 1][None])
    a0, a1 = _sc_layer(h, cs[2], src, dst)
    p, q = _tc_update_last(nf, a0, a1, uwa[2], uwb[2], upd_b[2][None],
                           dec_w[:D], dec_w[D:], dec_b[None])
    return _sc_decode(p, q, src, dst).T


# speedup vs baseline: 7.6797x; 1.0312x over previous
import functools

import jax
import jax.numpy as jnp
from jax import lax
from jax.experimental import pallas as pl
from jax.experimental.pallas import tpu as pltpu
from jax.experimental.pallas import tpu_sc as plsc

N = 10000
E = 320000
D = 128
DE = 16
NC = 2
NS = 16
NW = NC * NS
EPW = E // NW
CHL = 40
NSTEPL = EPW // CHL
CHD = 400
CHDS = 80
CHDK = CHD // CHDS
NSTEPD = EPW // CHD
RBASE = 624
ZR = 128
ZCOPIES = 5
LANES = 16

_mesh = plsc.VectorSubcoreMesh(core_axis_name="c", subcore_axis_name="s")


NBUF = 3


def _run_pipeline(nstep, process):
    process(0, 0, True, True, True)
    full = nstep - 4
    triples = full // 3

    def triple(k, carry):
        i = 3 * k + 1
        process(i, 1, True, True, False)
        process(i + 1, 2, True, True, False)
        process(i + 2, 0, True, True, False)
        return carry

    lax.fori_loop(0, triples, triple, None)
    for i in range(3 * triples + 1, nstep - 3):
        process(i, i % 3, True, True, False)
    process(nstep - 3, (nstep - 3) % 3, False, True, False)
    process(nstep - 2, (nstep - 2) % 3, False, False, False)
    process(nstep - 1, (nstep - 1) % 3, False, False, False)


@functools.partial(
    pl.kernel,
    out_type=(jax.ShapeDtypeStruct((N, D), jnp.float32),
              jax.ShapeDtypeStruct((N, D), jnp.float32)),
    mesh=_mesh,
    scratch_types=(
        pltpu.VMEM_SHARED((N, D), jnp.float32),
        pltpu.VMEM((NBUF, CHL), jnp.int32),
        pltpu.VMEM((NBUF, CHL), jnp.int32),
        pltpu.VMEM((NBUF, CHL, D), jnp.float32),
        pltpu.VMEM((NBUF, CHL, D // 2), jnp.int32),
    ) + (pltpu.SemaphoreType.DMA,) * (5 * NBUF),
    compiler_params=pltpu.CompilerParams(needs_layout_passes=False),
)
def _sc_layer(h_hbm, c_hbm, src_hbm, dst_hbm, out0_hbm, out1_hbm,
              aggr, sidx, didx, gbuf, cbuf, *sems):
    isem = sems[0:NBUF]
    dsem = sems[NBUF:2 * NBUF]
    csem = sems[2 * NBUF:3 * NBUF]
    gsem = sems[3 * NBUF:4 * NBUF]
    ssem = sems[4 * NBUF:5 * NBUF]

    c = lax.axis_index("c")
    s = lax.axis_index("s")
    wid = s * NC + c
    ebase = wid * EPW

    zero = jnp.zeros((LANES,), jnp.float32)

    def zrow(e, carry):
        for j in range(D // LANES):
            gbuf[0, e, pl.ds(j * LANES, LANES)] = zero
        return carry

    lax.fori_loop(0, CHL, zrow, None)
    for k in range(16):
        pltpu.sync_copy(gbuf.at[0], aggr.at[pl.ds(s * RBASE + k * CHL, CHL)])
    plsc.subcore_barrier()

    def issue_sidx(i, b):
        pltpu.async_copy(src_hbm.at[pl.ds(ebase + i * CHL, CHL)],
                         sidx.at[b], isem[b])

    def issue_didx(i, b):
        pltpu.async_copy(dst_hbm.at[pl.ds(ebase + i * CHL, CHL)],
                         didx.at[b], dsem[b])

    def issue_c(i, b):
        pltpu.async_copy(c_hbm.at[pl.ds(ebase + i * CHL, CHL)],
                         cbuf.at[b], csem[b])

    def wait_c(b):
        pltpu.make_async_copy(c_hbm.at[pl.ds(ebase, CHL)],
                              cbuf.at[b], csem[b]).wait()

    def issue_gather(b):
        pltpu.async_copy(h_hbm.at[sidx.at[b]], gbuf.at[b], gsem[b])

    def wait_gather(b):
        pltpu.make_async_copy(h_hbm.at[sidx.at[b]], gbuf.at[b],
                              gsem[b]).wait()

    def wait_lin(i, b, hbm, buf, sem):
        pltpu.make_async_copy(hbm.at[pl.ds(ebase + i * CHL, CHL)],
                              buf.at[b], sem[b]).wait()

    def issue_scatter(b):
        pltpu.async_copy(gbuf.at[b], aggr.at[didx.at[b]], ssem[b], add=True)

    def wait_scatter(b):
        pltpu.make_async_copy(gbuf.at[b], aggr.at[didx.at[b]],
                              ssem[b]).wait()

    def process(i, b, deep_pf, pf, first):
        wait_gather(b)
        wait_c(b)
        if deep_pf:
            issue_sidx(i + 3, b)

        def edge(e, carry):
            for j in range(D // 32):
                pair = plsc.bitcast(cbuf[b, e, pl.ds(LANES * j, LANES)],
                                    jnp.bfloat16)
                lo, hi = plsc.unpack(pair,
                                     format=plsc.PackFormat.INTERLEAVED)
                sl0 = pl.ds(32 * j, LANES)
                sl1 = pl.ds(32 * j + LANES, LANES)
                gbuf[b, e, sl0] = jnp.maximum(gbuf[b, e, sl0] + lo, 0.0)
                gbuf[b, e, sl1] = jnp.maximum(gbuf[b, e, sl1] + hi, 0.0)
            return carry

        lax.fori_loop(0, CHL, edge, None)
        wait_lin(i, b, dst_hbm, didx, dsem)
        issue_scatter(b)
        if pf:
            b2 = (b + 2) % NBUF
            if not first:
                wait_scatter(b2)
            issue_didx(i + 2, b2)
            issue_c(i + 2, b2)
            pltpu.make_async_copy(src_hbm.at[pl.ds(ebase, CHL)],
                                  sidx.at[b2], isem[b2]).wait()
            issue_gather(b2)

    for i in (0, 1):
        issue_sidx(i, i)
        issue_didx(i, i)
        issue_c(i, i)
        pltpu.make_async_copy(src_hbm.at[pl.ds(ebase, CHL)],
                              sidx.at[i], isem[i]).wait()
        issue_gather(i)
    issue_sidx(2, 2)

    _run_pipeline(NSTEPL, process)

    for b in range(NBUF):
        wait_scatter(b)

    plsc.subcore_barrier()
    for k in range(ZCOPIES):
        r0 = s * RBASE + k * ZR

        @pl.when(c == 0)
        def _():
            pltpu.sync_copy(aggr.at[pl.ds(r0, ZR)], out0_hbm.at[pl.ds(r0, ZR)])

        @pl.when(c == 1)
        def _():
            pltpu.sync_copy(aggr.at[pl.ds(r0, ZR)], out1_hbm.at[pl.ds(r0, ZR)])


@functools.partial(
    pl.kernel,
    out_type=jax.ShapeDtypeStruct((DE, E), jnp.float32),
    mesh=_mesh,
    scratch_types=(
        pltpu.VMEM((NBUF, CHDK, CHDS), jnp.int32),
        pltpu.VMEM((NBUF, CHDK, CHDS), jnp.int32),
        pltpu.VMEM((NBUF, CHD, DE), jnp.float32),
        pltpu.VMEM((NBUF, CHD, DE), jnp.float32),
        pltpu.VMEM((NBUF, DE, CHD), jnp.float32),
    ) + (pltpu.SemaphoreType.DMA,) * (5 * NBUF),
    compiler_params=pltpu.CompilerParams(use_tc_tiling_on_sc=False,
                                         needs_layout_passes=False),
)
def _sc_decode(p_hbm, q_hbm, src_hbm, dst_hbm, out_hbm,
               sidx, didx, pbuf, qbuf, wbuf, *sems):
    isem = sems[0:NBUF]
    dsem = sems[NBUF:2 * NBUF]
    psem = sems[2 * NBUF:3 * NBUF]
    qsem = sems[3 * NBUF:4 * NBUF]
    wsem = sems[4 * NBUF:5 * NBUF]

    c = lax.axis_index("c")
    s = lax.axis_index("s")
    ebase = (s * NC + c) * EPW
    rbase = (s * NC + c) * (EPW // CHDS)
    iota16 = lax.iota(jnp.int32, LANES)

    def issue_idx(i, b):
        pltpu.async_copy(src_hbm.at[pl.ds(rbase + i * CHDK, CHDK)],
                         sidx.at[b], isem[b])
        pltpu.async_copy(dst_hbm.at[pl.ds(rbase + i * CHDK, CHDK)],
                         didx.at[b], dsem[b])

    def wait_idx(b):
        pltpu.make_async_copy(src_hbm.at[pl.ds(rbase, CHDK)],
                              sidx.at[b], isem[b]).wait()
        pltpu.make_async_copy(dst_hbm.at[pl.ds(rbase, CHDK)],
                              didx.at[b], dsem[b]).wait()

    def issue_gathers(b):
        for k in range(CHDK):
            pltpu.async_copy(p_hbm.at[sidx.at[b, k]],
                             pbuf.at[b, pl.ds(k * CHDS, CHDS)], psem[b])
            pltpu.async_copy(q_hbm.at[didx.at[b, k]],
                             qbuf.at[b, pl.ds(k * CHDS, CHDS)], qsem[b])

    def wait_gathers(b):
        for k in range(CHDK):
            pltpu.make_async_copy(p_hbm.at[sidx.at[b, k]],
                                  pbuf.at[b, pl.ds(k * CHDS, CHDS)],
                                  psem[b]).wait()
            pltpu.make_async_copy(q_hbm.at[didx.at[b, k]],
                                  qbuf.at[b, pl.ds(k * CHDS, CHDS)],
                                  qsem[b]).wait()

    def wait_write(b):
        pltpu.make_async_copy(wbuf.at[b], out_hbm.at[:, pl.ds(ebase, CHD)],
                              wsem[b]).wait()

    def process(i, b, deep_pf, pf, first):
        wait_gathers(b)
        if deep_pf:
            issue_idx(i + 3, b)

        def edge(e, carry):
            v = pbuf[b, e, pl.ds(0, LANES)] + qbuf[b, e, pl.ds(0, LANES)]
            plsc.store_scatter(wbuf.at[b],
                               [iota16, jnp.full((LANES,), e, jnp.int32)], v)
            return carry

        lax.fori_loop(0, CHD, edge, None)
        pltpu.async_copy(wbuf.at[b],
                         out_hbm.at[:, pl.ds(ebase + i * CHD, CHD)], wsem[b])
        if pf:
            b2 = (b + 2) % NBUF
            if not first:
                wait_write(b2)
            wait_idx(b2)
            issue_gathers(b2)

    for i in (0, 1):
        issue_idx(i, i)
        wait_idx(i)
        issue_gathers(i)
    issue_idx(2, 2)

    _run_pipeline(NSTEPD, process)

    for b in range(NBUF):
        wait_write(b)


def _enc_body(x_ref, w_ref, b_ref, w1_ref, hb_ref, nf_ref, h_ref):
    nf = jnp.dot(x_ref[...], w_ref[...], preferred_element_type=jnp.float32)
    nf = nf + b_ref[...]
    nf_ref[...] = nf
    h_ref[...] = jnp.dot(nf, w1_ref[...],
                         preferred_element_type=jnp.float32) + hb_ref[...]


def _tc_encode(x, w, b, w1, hb):
    return pl.pallas_call(
        _enc_body,
        out_shape=(jax.ShapeDtypeStruct((N, D), jnp.float32),
                   jax.ShapeDtypeStruct((N, D), jnp.float32)),
    )(x, w, b, w1, hb)


CBLK = 6400


def _rhu_bf16_bits(x):
    return jax.lax.bitcast_convert_type(x, jnp.uint32) + jnp.uint32(0x8000)


def _cmul_body(eat_ref, we_ref, wo_ref, c_ref):
    dn = (((0,), (0,)), ((), ()))
    re = jax.lax.dot_general(eat_ref[...], we_ref[...], dn,
                             preferred_element_type=jnp.float32)
    ro = jax.lax.dot_general(eat_ref[...], wo_ref[...], dn,
                             preferred_element_type=jnp.float32)
    packed = (_rhu_bf16_bits(re) >> 16) | (_rhu_bf16_bits(ro)
                                           & jnp.uint32(0xFFFF0000))
    c_ref[...] = jax.lax.bitcast_convert_type(packed, jnp.int32)


def _tc_cmul_layer(ea_t, w_even, w_odd):
    h = D // 2
    return pl.pallas_call(
        _cmul_body,
        grid=(E // CBLK,),
        in_specs=[
            pl.BlockSpec((DE, CBLK), lambda i: (0, i)),
            pl.BlockSpec((DE, h), lambda i: (0, 0)),
            pl.BlockSpec((DE, h), lambda i: (0, 0)),
        ],
        out_specs=pl.BlockSpec((CBLK, h), lambda i: (i, 0)),
        out_shape=jax.ShapeDtypeStruct((E, h), jnp.int32),
    )(ea_t, w_even, w_odd)


def _upd_body(nf_ref, a0_ref, a1_ref, uwa_ref, uwb_ref, ub_ref, w1_ref, hb_ref,
              nf2_ref, h_ref):
    nf = nf_ref[...]
    ag = a0_ref[...] + a1_ref[...]
    u = jnp.dot(nf, uwa_ref[...], preferred_element_type=jnp.float32)
    u = u + jnp.dot(ag, uwb_ref[...], preferred_element_type=jnp.float32)
    u = jnp.maximum(u + ub_ref[...], 0.0)
    nf2 = nf + u
    nf2_ref[...] = nf2
    h_ref[...] = jnp.dot(nf2, w1_ref[...],
                         preferred_element_type=jnp.float32) + hb_ref[...]


def _tc_update(nf, a0, a1, uwa, uwb, ub, w1, hb):
    return pl.pallas_call(
        _upd_body,
        out_shape=(jax.ShapeDtypeStruct((N, D), jnp.float32),
                   jax.ShapeDtypeStruct((N, D), jnp.float32)),
    )(nf, a0, a1, uwa, uwb, ub, w1, hb)


def _updlast_body(nf_ref, a0_ref, a1_ref, uwa_ref, uwb_ref, ub_ref,
                  dwa_ref, dwb_ref, db_ref, p_ref, q_ref):
    nf = nf_ref[...]
    ag = a0_ref[...] + a1_ref[...]
    u = jnp.dot(nf, uwa_ref[...], preferred_element_type=jnp.float32)
    u = u + jnp.dot(ag, uwb_ref[...], preferred_element_type=jnp.float32)
    u = jnp.maximum(u + ub_ref[...], 0.0)
    nf2 = nf + u
    p_ref[...] = jnp.dot(nf2, dwa_ref[...],
                         preferred_element_type=jnp.float32) + db_ref[...]
    q_ref[...] = jnp.dot(nf2, dwb_ref[...], preferred_element_type=jnp.float32)


def _tc_update_last(nf, a0, a1, uwa, uwb, ub, dwa, dwb, db):
    return pl.pallas_call(
        _updlast_body,
        out_shape=(jax.ShapeDtypeStruct((N, DE), jnp.float32),
                   jax.ShapeDtypeStruct((N, DE), jnp.float32)),
    )(nf, a0, a1, uwa, uwb, ub, dwa, dwb, db)


def kernel(x, edge_index, edge_attr, node_enc_w, node_enc_b, edge_enc_w,
           edge_enc_b, dec_w, dec_b, msg_w, msg_b, upd_w, upd_b):
    src = edge_index[0].astype(jnp.int32)
    dst = edge_index[1].astype(jnp.int32)

    w1 = msg_w[:, :D, :]
    w2 = jnp.einsum("ef,lfm->lem", edge_enc_w,
                    msg_w[:, D:, :])
    hb = jnp.einsum("f,lfm->lm", edge_enc_b,
                    msg_w[:, D:, :]) + msg_b
    w2cat = jnp.concatenate([w2[0], w2[1], w2[2]], axis=1)
    col = jnp.arange(3 * D // 2)
    t32, k16 = col // LANES * 32, col % LANES
    w_even = w2cat[:, t32 + k16]
    w_odd = w2cat[:, t32 + LANES + k16]
    uwa = upd_w[:, :D, :]
    uwb = upd_w[:, D:, :]

    nf, h = _tc_encode(x, node_enc_w, node_enc_b[None], w1[0], hb[0][None])
    ea_t = edge_attr.T
    hw = 3 * D // 2
    cs = [_tc_cmul_layer(ea_t, w_even[:, l * hw // 3:(l + 1) * hw // 3],
                         w_odd[:, l * hw // 3:(l + 1) * hw // 3])
          for l in range(3)]

    for l in range(2):
        a0, a1 = _sc_layer(h, cs[l], src, dst)
        nf, h = _tc_update(nf, a0, a1, uwa[l], uwb[l], upd_b[l][None],
                           w1[l + 1], hb[l + 1][None])
    a0, a1 = _sc_layer(h, cs[2], src, dst)
    p, q = _tc_update_last(nf, a0, a1, uwa[2], uwb[2], upd_b[2][None],
                           dec_w[:D], dec_w[D:], dec_b[None])
    src2 = src.reshape(E // CHDS, CHDS)
    dst2 = dst.reshape(E // CHDS, CHDS)
    return _sc_decode(p, q, src2, dst2).T
